# baseline (speedup 1.0000x reference)
import functools
import jax
import jax.numpy as jnp
from jax import lax
from jax.experimental import pallas as pl
from jax.experimental.pallas import tpu as pltpu
from jax.experimental.pallas import tpu_sc as plsc

_BS = 64
_T = 2048
_D = 256
_NC = 2
_NS = 16
_NW = _NC * _NS
_BPW = _BS // _NW
_ROWS = _T // _NS


def _sc_body(table_hbm, out_hbm, spmem, sem):
    cid = lax.axis_index("c")
    sid = lax.axis_index("s")

    r0 = sid * _ROWS
    pltpu.sync_copy(table_hbm.at[pl.ds(r0, _ROWS)], spmem.at[pl.ds(r0, _ROWS)])
    plsc.subcore_barrier()

    base = (sid * _NC + cid) * _BPW
    copies = [
        pltpu.async_copy(spmem, out_hbm.at[base + j], sem)
        for j in range(_BPW)
    ]
    for c in copies:
        c.wait()


def kernel(mask, embed_weight):
    bs, t = mask.shape
    n_embed, d = embed_weight.shape

    mesh = plsc.VectorSubcoreMesh(core_axis_name="c", subcore_axis_name="s")
    k = functools.partial(
        pl.kernel,
        mesh=mesh,
        out_type=jax.ShapeDtypeStruct((bs, t, d), embed_weight.dtype),
        scratch_types=[
            pltpu.VMEM_SHARED((t, d), embed_weight.dtype),
            pltpu.SemaphoreType.DMA,
        ],
    )(_sc_body)
    return k(embed_weight[:t])
